# Initial kernel scaffold; baseline (speedup 1.0000x reference)
import jax
import jax.numpy as jnp
from jax.experimental import pallas as pl


def kernel(x, XY0, idx0, XY1, idx1, XY2, idx2, XY3, idx3, XY4, idx4, XY5, idx5):
    raise NotImplementedError("write your pallas kernel here")


# speedup vs baseline: 61.9232x; 61.9232x over previous
import functools

import jax
import jax.numpy as jnp
from jax import lax
from jax.experimental import pallas as pl
from jax.experimental.pallas import tpu as pltpu
from jax.experimental.pallas import tpu_sc as plsc

L = 256
HW = 512 * 1024
NW = 32
CHUNK = HW // NW
S = 512
NSUB = CHUNK // S
IB = 128
NB = S // IB


def _body(xt_h, gx_h, gy_h, fb_h, oi_h, out_h,
          gx_v, gy_v, fb_v, w00_v, w01_v, w10_v, w11_v,
          r00_v, r01_v, r10_v, r11_v, oi_v,
          g00_v, g01_v, g10_v, g11_v, out_v, gsem, ssem):
    cid = lax.axis_index("c")
    sid = lax.axis_index("s")
    wid = sid * 2 + cid
    base = wid * CHUNK

    def sub(t, carry):
        gbase = base + t * S
        pltpu.sync_copy(gx_h.at[pl.ds(gbase, S)], gx_v)
        pltpu.sync_copy(gy_h.at[pl.ds(gbase, S)], gy_v)
        pltpu.sync_copy(fb_h.at[pl.ds(gbase, S)], fb_v)
        for j in range(NB):
            pltpu.sync_copy(oi_h.at[pl.ds(gbase + j * IB, IB)], oi_v.at[j])

        for j in range(NB):
            for i in range(IB // 16):
                sl = pl.ds(j * IB + i * 16, 16)
                cs = pl.ds(i * 16, 16)
                gxv = gx_v[sl]
                gyv = gy_v[sl]
                x0 = gxv.astype(jnp.int32)
                y0 = gyv.astype(jnp.int32)
                wx = gxv - x0.astype(jnp.float32)
                wy = gyv - y0.astype(jnp.float32)
                y1 = jnp.minimum(y0 + 1, L - 1)
                r0 = fb_v[sl] + y0 * L + x0
                r1 = fb_v[sl] + y1 * L + x0
                r00_v[j, cs] = r0
                r01_v[j, cs] = r0 + 1
                r10_v[j, cs] = r1
                r11_v[j, cs] = r1 + 1
                w00_v[sl] = (1.0 - wx) * (1.0 - wy)
                w01_v[sl] = wx * (1.0 - wy)
                w10_v[sl] = (1.0 - wx) * wy
                w11_v[sl] = wx * wy

        cps = []
        for j in range(NB):
            ds = pl.ds(j * IB, IB)
            cps.append(pltpu.async_copy(xt_h.at[r00_v.at[j]], g00_v.at[ds], gsem))
            cps.append(pltpu.async_copy(xt_h.at[r01_v.at[j]], g01_v.at[ds], gsem))
            cps.append(pltpu.async_copy(xt_h.at[r10_v.at[j]], g10_v.at[ds], gsem))
            cps.append(pltpu.async_copy(xt_h.at[r11_v.at[j]], g11_v.at[ds], gsem))
        for cp in cps:
            cp.wait()

        def comb(g, c2):
            sl = pl.ds(g * 16, 16)
            wa = w00_v[sl]
            wb = w01_v[sl]
            wc = w10_v[sl]
            wd = w11_v[sl]
            lo = pl.ds(0, 16)
            hi = pl.ds(16, 16)
            for k in range(16):
                e = g * 16 + k
                out_v[e, lo] = (g00_v[e, lo] * wa[k] + g01_v[e, lo] * wb[k]
                                + g10_v[e, lo] * wc[k] + g11_v[e, lo] * wd[k])
                out_v[e, hi] = (g00_v[e, hi] * wa[k] + g01_v[e, hi] * wb[k]
                                + g10_v[e, hi] * wc[k] + g11_v[e, hi] * wd[k])
            return c2
        lax.fori_loop(0, S // 16, comb, 0)

        scps = []
        for j in range(NB):
            scps.append(pltpu.async_copy(out_v.at[pl.ds(j * IB, IB)],
                                         out_h.at[oi_v.at[j]], ssem))
        for cp in scps:
            cp.wait()
        return carry

    lax.fori_loop(0, NSUB, sub, 0)


@jax.jit
def _run(xt, gx, gy, fb, oi):
    mesh = plsc.VectorSubcoreMesh(core_axis_name="c", subcore_axis_name="s")
    f = functools.partial(
        pl.kernel,
        out_type=jax.ShapeDtypeStruct((HW, 32), jnp.float32),
        mesh=mesh,
        compiler_params=pltpu.CompilerParams(use_tc_tiling_on_sc=False),
        scratch_types=[
            pltpu.VMEM((S,), jnp.float32),
            pltpu.VMEM((S,), jnp.float32),
            pltpu.VMEM((S,), jnp.int32),
            pltpu.VMEM((S,), jnp.float32),
            pltpu.VMEM((S,), jnp.float32),
            pltpu.VMEM((S,), jnp.float32),
            pltpu.VMEM((S,), jnp.float32),
            pltpu.VMEM((NB, IB), jnp.int32),
            pltpu.VMEM((NB, IB), jnp.int32),
            pltpu.VMEM((NB, IB), jnp.int32),
            pltpu.VMEM((NB, IB), jnp.int32),
            pltpu.VMEM((NB, IB), jnp.int32),
            pltpu.VMEM((S, 32), jnp.float32),
            pltpu.VMEM((S, 32), jnp.float32),
            pltpu.VMEM((S, 32), jnp.float32),
            pltpu.VMEM((S, 32), jnp.float32),
            pltpu.VMEM((S, 32), jnp.float32),
            pltpu.SemaphoreType.DMA,
            pltpu.SemaphoreType.DMA,
        ],
    )(_body)
    return f(xt, gx, gy, fb, oi)


def kernel(x, XY0, idx0, XY1, idx1, XY2, idx2, XY3, idx3, XY4, idx4, XY5, idx5):
    XYs = [XY0, XY1, XY2, XY3, XY4, XY5]
    idxs = [idx0, idx1, idx2, idx3, idx4, idx5]
    equ = x.shape[0] // 6
    C = x.shape[1]
    xt = x.reshape(equ, 6, C, L * L).transpose(1, 3, 0, 2).reshape(6 * L * L, equ * C)
    xt = jnp.pad(xt, ((0, 8), (0, 0)))
    gx = jnp.concatenate([xy[:, 0] for xy in XYs])
    gy = jnp.concatenate([xy[:, 1] for xy in XYs])
    fb = jnp.concatenate([jnp.full((idxs[f].shape[0],), f * L * L, jnp.int32)
                          for f in range(6)])
    oi = jnp.concatenate(idxs)
    out_pm = _run(xt, gx, gy, fb, oi)
    return out_pm.reshape(HW, equ, C).transpose(1, 2, 0).reshape(equ, C, 512, 1024)


# speedup vs baseline: 78.5686x; 1.2688x over previous
import functools

import jax
import jax.numpy as jnp
from jax import lax
from jax.experimental import pallas as pl
from jax.experimental.pallas import tpu as pltpu
from jax.experimental.pallas import tpu_sc as plsc

L = 256
HW = 512 * 1024
R = 6 * L * L
NW = 32
CHUNK = HW // NW
S = 256
NSUB = CHUNK // S
IB = 128
NB = S // IB


def _body(xt_h, geom_h, oiz_h, out_h,
          oi_v, geomA, geomB, rA, rB, wA, wB, gA, gB, outA, outB,
          isem, gsemA, gsemB, ssemA, ssemB):
    cid = lax.axis_index("c")
    sid = lax.axis_index("s")
    wid = sid * 2 + cid
    base = wid * CHUNK
    mbase = wid * NSUB

    geom = [geomA, geomB]
    rv = [rA, rB]
    wv = [wA, wB]
    gv = [gA, gB]
    ov = [outA, outB]
    gsem = [gsemA, gsemB]
    ssem = [ssemA, ssemB]

    def drain(dst, sem, src):
        pltpu.make_async_copy(src, dst, sem).wait()

    def addr(p):
        def one(q, c2):
            cs = pl.ds(q * 16, 16)
            gxv = geom[p][0, cs]
            gyv = geom[p][1, cs]
            fbv = geom[p][2, cs].astype(jnp.int32)
            x0 = gxv.astype(jnp.int32)
            y0 = gyv.astype(jnp.int32)
            wx = gxv - x0.astype(jnp.float32)
            wy = gyv - y0.astype(jnp.float32)
            y1 = jnp.minimum(y0 + 1, L - 1)
            r0 = fbv + y0 * L + x0
            r1 = fbv + y1 * L + x0
            rv[p][0, cs] = r0
            rv[p][1, cs] = jnp.minimum(r0 + 1, R - 1)
            rv[p][2, cs] = r1
            rv[p][3, cs] = jnp.minimum(r1 + 1, R - 1)
            wv[p][0, cs] = (1.0 - wx) * (1.0 - wy)
            wv[p][1, cs] = wx * (1.0 - wy)
            wv[p][2, cs] = (1.0 - wx) * wy
            wv[p][3, cs] = wx * wy
            return c2
        lax.fori_loop(0, S // 16, one, 0)

    def fire_gathers(p):
        for tap in range(4):
            for j in range(NB):
                ds = pl.ds(j * IB, IB)
                pltpu.async_copy(xt_h.at[rv[p].at[tap, ds]],
                                 gv[p].at[tap, ds], gsem[p])

    def drain_gathers(p):
        for tap in range(4):
            drain(gv[p].at[tap], gsem[p], xt_h.at[pl.ds(0, S)])

    def combine(p):
        def one(g, c2):
            sl = pl.ds(g * 16, 16)
            wa = wv[p][0, sl]
            wb = wv[p][1, sl]
            wc = wv[p][2, sl]
            wd = wv[p][3, sl]
            lo = pl.ds(0, 16)
            hi = pl.ds(16, 16)
            for k in range(16):
                e = g * 16 + k
                ov[p][e, lo] = (gv[p][0, e, lo] * wa[k] + gv[p][1, e, lo] * wb[k]
                                + gv[p][2, e, lo] * wc[k] + gv[p][3, e, lo] * wd[k])
                ov[p][e, hi] = (gv[p][0, e, hi] * wa[k] + gv[p][1, e, hi] * wb[k]
                                + gv[p][2, e, hi] * wc[k] + gv[p][3, e, hi] * wd[k])
            return c2
        lax.fori_loop(0, S // 16, one, 0)

    def phase(t, par, drain_scatter):
        nxt = 1 - par
        drain(geom[nxt], isem, geom_h.at[0])
        addr(nxt)
        fire_gathers(nxt)
        pltpu.async_copy(geom_h.at[mbase + t + 2], geom[par], isem)
        if drain_scatter:
            drain(ov[par], ssem[par], xt_h.at[pl.ds(0, S)])
        drain_gathers(par)
        combine(par)
        for j in range(NB):
            pltpu.async_copy(ov[par].at[pl.ds(j * IB, IB)],
                             out_h.at[oi_v.at[t * NB + j]], ssem[par])

    pltpu.sync_copy(oiz_h.at[pl.ds(wid * (CHUNK // IB), CHUNK // IB)], oi_v)
    pltpu.sync_copy(geom_h.at[mbase], geomA)
    addr(0)
    fire_gathers(0)
    pltpu.async_copy(geom_h.at[mbase + 1], geomB, isem)
    phase(0, 0, False)
    phase(1, 1, False)

    def step(k, c2):
        t = 2 * k + 2
        phase(t, 0, True)
        phase(t + 1, 1, True)
        return c2
    lax.fori_loop(0, (NSUB - 2) // 2, step, 0)

    drain_gathers(0)
    drain(geom[1], isem, geom_h.at[0])
    drain(ov[0], ssem[0], xt_h.at[pl.ds(0, S)])
    drain(ov[1], ssem[1], xt_h.at[pl.ds(0, S)])


@jax.jit
def _run(xt, geom, oiz):
    mesh = plsc.VectorSubcoreMesh(core_axis_name="c", subcore_axis_name="s")
    f = functools.partial(
        pl.kernel,
        out_type=jax.ShapeDtypeStruct((HW, 32), jnp.float32),
        mesh=mesh,
        compiler_params=pltpu.CompilerParams(use_tc_tiling_on_sc=False),
        scratch_types=[
            pltpu.VMEM((CHUNK // IB, IB), jnp.int32),
            pltpu.VMEM((3, S), jnp.float32),
            pltpu.VMEM((3, S), jnp.float32),
            pltpu.VMEM((4, S), jnp.int32),
            pltpu.VMEM((4, S), jnp.int32),
            pltpu.VMEM((4, S), jnp.float32),
            pltpu.VMEM((4, S), jnp.float32),
            pltpu.VMEM((4, S, 32), jnp.float32),
            pltpu.VMEM((4, S, 32), jnp.float32),
            pltpu.VMEM((S, 32), jnp.float32),
            pltpu.VMEM((S, 32), jnp.float32),
            pltpu.SemaphoreType.DMA,
            pltpu.SemaphoreType.DMA,
            pltpu.SemaphoreType.DMA,
            pltpu.SemaphoreType.DMA,
            pltpu.SemaphoreType.DMA,
        ],
    )(_body)
    return f(xt, geom, oiz)


def kernel(x, XY0, idx0, XY1, idx1, XY2, idx2, XY3, idx3, XY4, idx4, XY5, idx5):
    XYs = [XY0, XY1, XY2, XY3, XY4, XY5]
    idxs = [idx0, idx1, idx2, idx3, idx4, idx5]
    equ = x.shape[0] // 6
    C = x.shape[1]
    xt = x.reshape(equ, 6, C, L * L).transpose(1, 3, 0, 2).reshape(R, equ * C)
    gx = jnp.concatenate([xy[:, 0] for xy in XYs])
    gy = jnp.concatenate([xy[:, 1] for xy in XYs])
    fb = jnp.concatenate([jnp.full((idxs[f].shape[0],), f * L * L, jnp.int32)
                          for f in range(6)])
    oi = jnp.concatenate(idxs)
    geom = jnp.stack([gx, gy, fb.astype(jnp.float32)])
    geom = geom.reshape(3, HW // S, S).transpose(1, 0, 2)
    geom = jnp.pad(geom, ((0, 2), (0, 0), (0, 0)))
    oiz = oi.reshape(HW // IB, IB)
    out_pm = _run(xt, geom, oiz)
    return out_pm.reshape(HW, equ, C).transpose(1, 2, 0).reshape(equ, C, 512, 1024)
